# initial kernel scaffold (unmeasured)
import jax
import jax.numpy as jnp
from jax import lax
from jax.experimental import pallas as pl
from jax.experimental.pallas import tpu as pltpu

T = 1024
D = 1024
E = 8
F = 2048
XS = 2
TPER = T // XS
EPER = E // XS


def _xnbr():
    my_x = lax.axis_index("x")
    my_y = lax.axis_index("y")
    return my_x, my_y, (1 - my_x, my_y)


def _exchange_body(x_ref, rt_ref, xfull_ref, rtfull_ref, send_sems, recv_sems):
    my_x, _, nbr = _xnbr()

    barrier = pltpu.get_barrier_semaphore()
    pl.semaphore_signal(barrier, inc=1, device_id=nbr,
                        device_id_type=pl.DeviceIdType.MESH)
    pl.semaphore_wait(barrier, 1)

    xfull_ref[pl.ds(my_x * TPER, TPER), :] = x_ref[...]
    rtfull_ref[pl.ds(my_x * EPER, EPER), :] = rt_ref[...]

    rx = pltpu.make_async_remote_copy(
        src_ref=x_ref,
        dst_ref=xfull_ref.at[pl.ds(my_x * TPER, TPER), :],
        send_sem=send_sems.at[0],
        recv_sem=recv_sems.at[0],
        device_id=nbr,
        device_id_type=pl.DeviceIdType.MESH,
    )
    rr = pltpu.make_async_remote_copy(
        src_ref=rt_ref,
        dst_ref=rtfull_ref.at[pl.ds(my_x * EPER, EPER), :],
        send_sem=send_sems.at[1],
        recv_sem=recv_sems.at[1],
        device_id=nbr,
        device_id_type=pl.DeviceIdType.MESH,
    )
    rx.start()
    rr.start()
    rx.wait()
    rr.wait()


def _exchange(x, router_t):
    return pl.pallas_call(
        _exchange_body,
        out_shape=(
            jax.ShapeDtypeStruct((T, D), jnp.float32),
            jax.ShapeDtypeStruct((E, D), jnp.float32),
        ),
        in_specs=[
            pl.BlockSpec(memory_space=pltpu.VMEM),
            pl.BlockSpec(memory_space=pltpu.VMEM),
        ],
        out_specs=(
            pl.BlockSpec(memory_space=pltpu.VMEM),
            pl.BlockSpec(memory_space=pltpu.VMEM),
        ),
        scratch_shapes=[
            pltpu.SemaphoreType.DMA((2,)),
            pltpu.SemaphoreType.DMA((2,)),
        ],
        compiler_params=pltpu.CompilerParams(collective_id=0),
    )(x, router_t)


def _ffn_body(x_ref, w1_ref, w2_ref, wt_ref, out_ref):
    e = pl.program_id(0)
    xb = x_ref[...]
    w1 = w1_ref[0].astype(jnp.bfloat16)
    h = jnp.dot(xb, w1, preferred_element_type=jnp.float32)
    h = jnp.maximum(h, 0.0).astype(jnp.bfloat16)
    w2 = w2_ref[0].astype(jnp.bfloat16)
    o = jnp.dot(h, w2, preferred_element_type=jnp.float32)
    o = o * wt_ref[...]

    @pl.when(e == 0)
    def _():
        out_ref[...] = o

    @pl.when(e != 0)
    def _():
        out_ref[...] += o


def _ffn(x_bf16, W1, W2, wloc):
    return pl.pallas_call(
        _ffn_body,
        grid=(EPER,),
        out_shape=jax.ShapeDtypeStruct((T, D), jnp.float32),
        in_specs=[
            pl.BlockSpec((T, D), lambda e: (0, 0)),
            pl.BlockSpec((1, D, F), lambda e: (e, 0, 0)),
            pl.BlockSpec((1, F, D), lambda e: (e, 0, 0)),
            pl.BlockSpec((T, 1), lambda e: (0, e)),
        ],
        out_specs=pl.BlockSpec((T, D), lambda e: (0, 0)),
        compiler_params=pltpu.CompilerParams(
            dimension_semantics=("arbitrary",),
        ),
    )(x_bf16, W1, W2, wloc)


def _combine_body(part_ref, out_ref, recv_buf, send_sem, recv_sem):
    my_x, _, nbr = _xnbr()
    other = 1 - my_x

    barrier = pltpu.get_barrier_semaphore()
    pl.semaphore_signal(barrier, inc=1, device_id=nbr,
                        device_id_type=pl.DeviceIdType.MESH)
    pl.semaphore_wait(barrier, 1)

    rdma = pltpu.make_async_remote_copy(
        src_ref=part_ref.at[pl.ds(other * TPER, TPER), :],
        dst_ref=recv_buf,
        send_sem=send_sem,
        recv_sem=recv_sem,
        device_id=nbr,
        device_id_type=pl.DeviceIdType.MESH,
    )
    rdma.start()
    rdma.wait()

    out_ref[...] = part_ref[pl.ds(my_x * TPER, TPER), :] + recv_buf[...]


def _combine(part):
    return pl.pallas_call(
        _combine_body,
        out_shape=jax.ShapeDtypeStruct((TPER, D), jnp.float32),
        in_specs=[pl.BlockSpec(memory_space=pltpu.VMEM)],
        out_specs=pl.BlockSpec(memory_space=pltpu.VMEM),
        scratch_shapes=[
            pltpu.VMEM((TPER, D), jnp.float32),
            pltpu.SemaphoreType.DMA,
            pltpu.SemaphoreType.DMA,
        ],
        compiler_params=pltpu.CompilerParams(collective_id=1),
    )(part)


def kernel(x, router, W1, W2):
    my_x = lax.axis_index("x")

    xfull, rtfull = _exchange(x, router.T)

    gates = jax.lax.dot_general(
        xfull, rtfull,
        (((1,), (1,)), ((), ())),
        precision=jax.lax.Precision.HIGHEST,
    )
    top2v, top2i = jax.lax.top_k(gates, 2)
    m = jnp.max(top2v, axis=1, keepdims=True)
    w = jnp.exp(top2v - m)
    w = w / jnp.sum(w, axis=1, keepdims=True)
    eids = jnp.arange(E)[None, :]
    wtok = (w[:, 0:1] * (top2i[:, 0:1] == eids)
            + w[:, 1:2] * (top2i[:, 1:2] == eids)).astype(jnp.float32)
    wloc = lax.dynamic_slice(wtok, (0, my_x * EPER), (T, EPER))

    part = _ffn(xfull.astype(jnp.bfloat16), W1, W2, wloc)
    return _combine(part)


# baseline (device time: 114211 ns/iter reference)
import jax
import jax.numpy as jnp
from jax import lax
from jax.experimental import pallas as pl
from jax.experimental.pallas import tpu as pltpu

T = 1024
D = 1024
E = 8
F = 2048
XS = 2
TPER = T // XS
EPER = E // XS


def _xnbr():
    my_x = lax.axis_index("x")
    my_y = lax.axis_index("y")
    return my_x, my_y, (1 - my_x, my_y)


def _exchange_body(x_ref, rt_ref, xfull_ref, rtfull_ref, send_sems, recv_sems):
    my_x, _, nbr = _xnbr()

    barrier = pltpu.get_barrier_semaphore()
    pl.semaphore_signal(barrier, inc=1, device_id=nbr,
                        device_id_type=pl.DeviceIdType.MESH)
    pl.semaphore_wait(barrier, 1)

    xfull_ref[pl.ds(my_x * TPER, TPER), :] = x_ref[...]
    rtfull_ref[my_x] = rt_ref[...]

    rx = pltpu.make_async_remote_copy(
        src_ref=x_ref,
        dst_ref=xfull_ref.at[pl.ds(my_x * TPER, TPER), :],
        send_sem=send_sems.at[0],
        recv_sem=recv_sems.at[0],
        device_id=nbr,
        device_id_type=pl.DeviceIdType.MESH,
    )
    rr = pltpu.make_async_remote_copy(
        src_ref=rt_ref,
        dst_ref=rtfull_ref.at[my_x],
        send_sem=send_sems.at[1],
        recv_sem=recv_sems.at[1],
        device_id=nbr,
        device_id_type=pl.DeviceIdType.MESH,
    )
    rx.start()
    rr.start()
    rx.wait()
    rr.wait()


def _exchange(x, router_t):
    return pl.pallas_call(
        _exchange_body,
        out_shape=(
            jax.ShapeDtypeStruct((T, D), jnp.float32),
            jax.ShapeDtypeStruct((XS, EPER, D), jnp.float32),
        ),
        in_specs=[
            pl.BlockSpec(memory_space=pltpu.VMEM),
            pl.BlockSpec(memory_space=pltpu.VMEM),
        ],
        out_specs=(
            pl.BlockSpec(memory_space=pltpu.VMEM),
            pl.BlockSpec(memory_space=pltpu.VMEM),
        ),
        scratch_shapes=[
            pltpu.SemaphoreType.DMA((2,)),
            pltpu.SemaphoreType.DMA((2,)),
        ],
        compiler_params=pltpu.CompilerParams(collective_id=0),
    )(x, router_t)


def _ffn_body(x_ref, w1_ref, w2_ref, wt_ref, out_ref):
    e = pl.program_id(0)
    xb = x_ref[...]
    w1 = w1_ref[0].astype(jnp.bfloat16)
    h = jnp.dot(xb, w1, preferred_element_type=jnp.float32)
    h = jnp.maximum(h, 0.0).astype(jnp.bfloat16)
    w2 = w2_ref[0].astype(jnp.bfloat16)
    o = jnp.dot(h, w2, preferred_element_type=jnp.float32)
    col = lax.broadcasted_iota(jnp.int32, (T, EPER), 1)
    wcol = jnp.sum(jnp.where(col == e, wt_ref[...], 0.0), axis=1,
                   keepdims=True)
    o = o * wcol

    @pl.when(e == 0)
    def _():
        out_ref[...] = o

    @pl.when(e != 0)
    def _():
        out_ref[...] += o


def _ffn(x_bf16, W1, W2, wloc):
    return pl.pallas_call(
        _ffn_body,
        grid=(EPER,),
        out_shape=jax.ShapeDtypeStruct((T, D), jnp.float32),
        in_specs=[
            pl.BlockSpec((T, D), lambda e: (0, 0)),
            pl.BlockSpec((1, D, F), lambda e: (e, 0, 0)),
            pl.BlockSpec((1, F, D), lambda e: (e, 0, 0)),
            pl.BlockSpec((T, EPER), lambda e: (0, 0)),
        ],
        out_specs=pl.BlockSpec((T, D), lambda e: (0, 0)),
        compiler_params=pltpu.CompilerParams(
            dimension_semantics=("arbitrary",),
            vmem_limit_bytes=100 * 1024 * 1024,
        ),
    )(x_bf16, W1, W2, wloc)


def _combine_body(part_ref, out_ref, recv_buf, send_sem, recv_sem):
    my_x, _, nbr = _xnbr()
    other = 1 - my_x

    barrier = pltpu.get_barrier_semaphore()
    pl.semaphore_signal(barrier, inc=1, device_id=nbr,
                        device_id_type=pl.DeviceIdType.MESH)
    pl.semaphore_wait(barrier, 1)

    rdma = pltpu.make_async_remote_copy(
        src_ref=part_ref.at[pl.ds(other * TPER, TPER), :],
        dst_ref=recv_buf,
        send_sem=send_sem,
        recv_sem=recv_sem,
        device_id=nbr,
        device_id_type=pl.DeviceIdType.MESH,
    )
    rdma.start()
    rdma.wait()

    out_ref[...] = part_ref[pl.ds(my_x * TPER, TPER), :] + recv_buf[...]


def _combine(part):
    return pl.pallas_call(
        _combine_body,
        out_shape=jax.ShapeDtypeStruct((TPER, D), jnp.float32),
        in_specs=[pl.BlockSpec(memory_space=pltpu.VMEM)],
        out_specs=pl.BlockSpec(memory_space=pltpu.VMEM),
        scratch_shapes=[
            pltpu.VMEM((TPER, D), jnp.float32),
            pltpu.SemaphoreType.DMA,
            pltpu.SemaphoreType.DMA,
        ],
        compiler_params=pltpu.CompilerParams(collective_id=1),
    )(part)


def kernel(x, router, W1, W2):
    my_x = lax.axis_index("x")

    xfull, rtfull = _exchange(x, router.T)
    rtfull = rtfull.reshape(E, D)

    gates = jax.lax.dot_general(
        xfull, rtfull,
        (((1,), (1,)), ((), ())),
        precision=jax.lax.Precision.HIGHEST,
    )
    top2v, top2i = jax.lax.top_k(gates, 2)
    m = jnp.max(top2v, axis=1, keepdims=True)
    w = jnp.exp(top2v - m)
    w = w / jnp.sum(w, axis=1, keepdims=True)
    eids = jnp.arange(E)[None, :]
    wtok = (w[:, 0:1] * (top2i[:, 0:1] == eids)
            + w[:, 1:2] * (top2i[:, 1:2] == eids)).astype(jnp.float32)
    wloc = lax.dynamic_slice(wtok, (0, my_x * EPER), (T, EPER))

    part = _ffn(xfull.astype(jnp.bfloat16), W1, W2, wloc)
    return _combine(part)


# device time: 87421 ns/iter; 1.3064x vs baseline; 1.3064x over previous
import jax
import jax.numpy as jnp
from jax import lax
from jax.experimental import pallas as pl
from jax.experimental.pallas import tpu as pltpu

T = 1024
D = 1024
E = 8
F = 2048
XS = 2
TPER = T // XS
EPER = E // XS
ELOC = EPER // 2


def _dispatch_body(x_ref, rt_ref, xbf_ref, gates_ref, rtfull_ref,
                   send_sems, recv_sems):
    my_x = lax.axis_index("x")
    my_y = lax.axis_index("y")
    xnbr = (1 - my_x, my_y)

    barrier = pltpu.get_barrier_semaphore()
    pl.semaphore_signal(barrier, inc=1, device_id=xnbr,
                        device_id_type=pl.DeviceIdType.MESH)
    pl.semaphore_wait(barrier, 1)

    rtfull_ref[my_x] = rt_ref[...]
    rdma_rt = pltpu.make_async_remote_copy(
        src_ref=rt_ref,
        dst_ref=rtfull_ref.at[my_x],
        send_sem=send_sems.at[0],
        recv_sem=recv_sems.at[0],
        device_id=xnbr,
        device_id_type=pl.DeviceIdType.MESH,
    )
    rdma_rt.start()

    xbf_ref[pl.ds(my_x * TPER, TPER), :] = x_ref[...].astype(jnp.bfloat16)
    rdma_x = pltpu.make_async_remote_copy(
        src_ref=xbf_ref.at[pl.ds(my_x * TPER, TPER), :],
        dst_ref=xbf_ref.at[pl.ds(my_x * TPER, TPER), :],
        send_sem=send_sems.at[1],
        recv_sem=recv_sems.at[1],
        device_id=xnbr,
        device_id_type=pl.DeviceIdType.MESH,
    )
    rdma_x.start()

    rdma_rt.wait()
    rt_all = rtfull_ref[...].reshape(E, D)
    gates_own = jax.lax.dot_general(
        x_ref[...], rt_all, (((1,), (1,)), ((), ())),
        preferred_element_type=jnp.float32,
        precision=jax.lax.Precision.HIGHEST,
    )
    gates_ref[pl.ds(my_x * TPER, TPER), :] = gates_own
    rdma_g = pltpu.make_async_remote_copy(
        src_ref=gates_ref.at[pl.ds(my_x * TPER, TPER), :],
        dst_ref=gates_ref.at[pl.ds(my_x * TPER, TPER), :],
        send_sem=send_sems.at[2],
        recv_sem=recv_sems.at[2],
        device_id=xnbr,
        device_id_type=pl.DeviceIdType.MESH,
    )
    rdma_g.start()

    rdma_g.wait()
    rdma_x.wait()


def _dispatch(x, router_t):
    return pl.pallas_call(
        _dispatch_body,
        out_shape=(
            jax.ShapeDtypeStruct((T, D), jnp.bfloat16),
            jax.ShapeDtypeStruct((T, E), jnp.float32),
        ),
        in_specs=[
            pl.BlockSpec(memory_space=pltpu.VMEM),
            pl.BlockSpec(memory_space=pltpu.VMEM),
        ],
        out_specs=(
            pl.BlockSpec(memory_space=pltpu.VMEM),
            pl.BlockSpec(memory_space=pltpu.VMEM),
        ),
        scratch_shapes=[
            pltpu.VMEM((XS, EPER, D), jnp.float32),
            pltpu.SemaphoreType.DMA((3,)),
            pltpu.SemaphoreType.DMA((3,)),
        ],
        compiler_params=pltpu.CompilerParams(collective_id=0),
    )(x, router_t)


def _ffn_body(off_ref, x_ref, w1_ref, w2_ref, wt_ref, out_ref):
    e = pl.program_id(0)
    xb = x_ref[...]
    w1 = w1_ref[0].astype(jnp.bfloat16)
    h = jnp.dot(xb, w1, preferred_element_type=jnp.float32)
    h = jnp.maximum(h, 0.0).astype(jnp.bfloat16)
    w2 = w2_ref[0].astype(jnp.bfloat16)
    o = jnp.dot(h, w2, preferred_element_type=jnp.float32)
    gid = off_ref[1] + e
    col = lax.broadcasted_iota(jnp.int32, (T, E), 1)
    wcol = jnp.sum(jnp.where(col == gid, wt_ref[...], 0.0), axis=1,
                   keepdims=True)
    o = o * wcol

    @pl.when(e == 0)
    def _():
        out_ref[...] = o

    @pl.when(e != 0)
    def _():
        out_ref[...] += o


def _ffn(offs, x_bf16, W1, W2, wtok):
    grid_spec = pltpu.PrefetchScalarGridSpec(
        num_scalar_prefetch=1,
        grid=(ELOC,),
        in_specs=[
            pl.BlockSpec((T, D), lambda e, off: (0, 0)),
            pl.BlockSpec((1, D, F), lambda e, off: (off[0] + e, 0, 0)),
            pl.BlockSpec((1, F, D), lambda e, off: (off[0] + e, 0, 0)),
            pl.BlockSpec((T, E), lambda e, off: (0, 0)),
        ],
        out_specs=pl.BlockSpec((T, D), lambda e, off: (0, 0)),
    )
    return pl.pallas_call(
        _ffn_body,
        grid_spec=grid_spec,
        out_shape=jax.ShapeDtypeStruct((T, D), jnp.float32),
        compiler_params=pltpu.CompilerParams(
            dimension_semantics=("arbitrary",),
            vmem_limit_bytes=100 * 1024 * 1024,
        ),
    )(offs, x_bf16, W1, W2, wtok)


def _combine_body(part_ref, out_ref, sx_ref, rx_ref, sy_ref, ry_ref,
                  send_sems, recv_sems):
    my_x = lax.axis_index("x")
    my_y = lax.axis_index("y")
    xnbr = (1 - my_x, my_y)
    ynbr = (my_x, 1 - my_y)
    other = 1 - my_x

    barrier = pltpu.get_barrier_semaphore()
    for nbr in (xnbr, ynbr):
        pl.semaphore_signal(barrier, inc=1, device_id=nbr,
                            device_id_type=pl.DeviceIdType.MESH)
    pl.semaphore_wait(barrier, 2)

    sx_ref[...] = part_ref[pl.ds(other * TPER, TPER), :].astype(jnp.bfloat16)
    rdma_x = pltpu.make_async_remote_copy(
        src_ref=sx_ref,
        dst_ref=rx_ref,
        send_sem=send_sems.at[0],
        recv_sem=recv_sems.at[0],
        device_id=xnbr,
        device_id_type=pl.DeviceIdType.MESH,
    )
    rdma_x.start()
    rdma_x.wait()

    s1 = part_ref[pl.ds(my_x * TPER, TPER), :] + rx_ref[...].astype(jnp.float32)

    sy_ref[...] = s1.astype(jnp.bfloat16)
    out_ref[...] = s1
    rdma_y = pltpu.make_async_remote_copy(
        src_ref=sy_ref,
        dst_ref=ry_ref,
        send_sem=send_sems.at[1],
        recv_sem=recv_sems.at[1],
        device_id=ynbr,
        device_id_type=pl.DeviceIdType.MESH,
    )
    rdma_y.start()
    rdma_y.wait()

    out_ref[...] += ry_ref[...].astype(jnp.float32)


def _combine(part):
    return pl.pallas_call(
        _combine_body,
        out_shape=jax.ShapeDtypeStruct((TPER, D), jnp.float32),
        in_specs=[pl.BlockSpec(memory_space=pltpu.VMEM)],
        out_specs=pl.BlockSpec(memory_space=pltpu.VMEM),
        scratch_shapes=[
            pltpu.VMEM((TPER, D), jnp.bfloat16),
            pltpu.VMEM((TPER, D), jnp.bfloat16),
            pltpu.VMEM((TPER, D), jnp.bfloat16),
            pltpu.VMEM((TPER, D), jnp.bfloat16),
            pltpu.SemaphoreType.DMA((2,)),
            pltpu.SemaphoreType.DMA((2,)),
        ],
        compiler_params=pltpu.CompilerParams(collective_id=1),
    )(part)


def kernel(x, router, W1, W2):
    my_x = lax.axis_index("x")
    my_y = lax.axis_index("y")

    xbf, gates = _dispatch(x, router.T)

    top2v, top2i = jax.lax.top_k(gates, 2)
    m = jnp.max(top2v, axis=1, keepdims=True)
    w = jnp.exp(top2v - m)
    w = w / jnp.sum(w, axis=1, keepdims=True)
    eids = jnp.arange(E)[None, :]
    wtok = (w[:, 0:1] * (top2i[:, 0:1] == eids)
            + w[:, 1:2] * (top2i[:, 1:2] == eids)).astype(jnp.float32)

    offs = jnp.stack([2 * my_y, 4 * my_x + 2 * my_y]).astype(jnp.int32)

    part = _ffn(offs, xbf, W1, W2, wtok)
    return _combine(part)


# device time: 75872 ns/iter; 1.5053x vs baseline; 1.1522x over previous
import jax
import jax.numpy as jnp
from jax import lax
from jax.experimental import pallas as pl
from jax.experimental.pallas import tpu as pltpu

T = 1024
D = 1024
E = 8
F = 2048
XS = 2
TPER = T // XS
EPER = E // XS
ELOC = EPER // 2


def _dispatch_body(x_ref, rt_ref, xbf_ref, gates_ref, rtfull_ref,
                   send_sems, recv_sems):
    my_x = lax.axis_index("x")
    my_y = lax.axis_index("y")
    xnbr = (1 - my_x, my_y)

    barrier = pltpu.get_barrier_semaphore()
    pl.semaphore_signal(barrier, inc=1, device_id=xnbr,
                        device_id_type=pl.DeviceIdType.MESH)
    pl.semaphore_wait(barrier, 1)

    rtfull_ref[my_x] = rt_ref[...]
    rdma_rt = pltpu.make_async_remote_copy(
        src_ref=rt_ref,
        dst_ref=rtfull_ref.at[my_x],
        send_sem=send_sems.at[0],
        recv_sem=recv_sems.at[0],
        device_id=xnbr,
        device_id_type=pl.DeviceIdType.MESH,
    )
    rdma_rt.start()

    xbf_ref[pl.ds(my_x * TPER, TPER), :] = x_ref[...].astype(jnp.bfloat16)
    rdma_x = pltpu.make_async_remote_copy(
        src_ref=xbf_ref.at[pl.ds(my_x * TPER, TPER), :],
        dst_ref=xbf_ref.at[pl.ds(my_x * TPER, TPER), :],
        send_sem=send_sems.at[1],
        recv_sem=recv_sems.at[1],
        device_id=xnbr,
        device_id_type=pl.DeviceIdType.MESH,
    )
    rdma_x.start()

    rdma_rt.wait()
    rt_all = rtfull_ref[...].reshape(E, D)
    gates_own = jax.lax.dot_general(
        x_ref[...], rt_all, (((1,), (1,)), ((), ())),
        preferred_element_type=jnp.float32,
        precision=jax.lax.Precision.HIGHEST,
    )
    gates_ref[pl.ds(my_x * TPER, TPER), :] = gates_own
    rdma_g = pltpu.make_async_remote_copy(
        src_ref=gates_ref.at[pl.ds(my_x * TPER, TPER), :],
        dst_ref=gates_ref.at[pl.ds(my_x * TPER, TPER), :],
        send_sem=send_sems.at[2],
        recv_sem=recv_sems.at[2],
        device_id=xnbr,
        device_id_type=pl.DeviceIdType.MESH,
    )
    rdma_g.start()

    rdma_g.wait()
    rdma_x.wait()


def _dispatch(x, router_t):
    return pl.pallas_call(
        _dispatch_body,
        out_shape=(
            jax.ShapeDtypeStruct((T, D), jnp.bfloat16),
            jax.ShapeDtypeStruct((T, E), jnp.float32),
        ),
        in_specs=[
            pl.BlockSpec(memory_space=pltpu.VMEM),
            pl.BlockSpec(memory_space=pltpu.VMEM),
        ],
        out_specs=(
            pl.BlockSpec(memory_space=pltpu.VMEM),
            pl.BlockSpec(memory_space=pltpu.VMEM),
        ),
        scratch_shapes=[
            pltpu.VMEM((XS, EPER, D), jnp.float32),
            pltpu.SemaphoreType.DMA((3,)),
            pltpu.SemaphoreType.DMA((3,)),
        ],
        compiler_params=pltpu.CompilerParams(collective_id=0),
    )(x, router_t)


FB = 2
FBLK = F // FB


def _ffn_body(off_ref, x_ref, w1_ref, w2_ref, wt_ref, out_ref):
    e = pl.program_id(0)
    fb = pl.program_id(1)
    xb = x_ref[...]
    w1 = w1_ref[0].astype(jnp.bfloat16)
    h = jnp.dot(xb, w1, preferred_element_type=jnp.float32)
    h = jnp.maximum(h, 0.0).astype(jnp.bfloat16)
    w2 = w2_ref[0].astype(jnp.bfloat16)
    o = jnp.dot(h, w2, preferred_element_type=jnp.float32)
    gid = off_ref[1] + e
    col = lax.broadcasted_iota(jnp.int32, (T, E), 1)
    wcol = jnp.sum(jnp.where(col == gid, wt_ref[...], 0.0), axis=1,
                   keepdims=True)
    o = o * wcol

    @pl.when((e == 0) & (fb == 0))
    def _():
        out_ref[...] = o

    @pl.when((e != 0) | (fb != 0))
    def _():
        out_ref[...] += o


def _ffn(offs, x_bf16, W1, W2, wtok):
    grid_spec = pltpu.PrefetchScalarGridSpec(
        num_scalar_prefetch=1,
        grid=(ELOC, FB),
        in_specs=[
            pl.BlockSpec((T, D), lambda e, fb, off: (0, 0)),
            pl.BlockSpec((1, D, FBLK), lambda e, fb, off: (off[0] + e, 0, fb)),
            pl.BlockSpec((1, FBLK, D), lambda e, fb, off: (off[0] + e, fb, 0)),
            pl.BlockSpec((T, E), lambda e, fb, off: (0, 0)),
        ],
        out_specs=pl.BlockSpec((T, D), lambda e, fb, off: (0, 0)),
    )
    return pl.pallas_call(
        _ffn_body,
        grid_spec=grid_spec,
        out_shape=jax.ShapeDtypeStruct((T, D), jnp.float32),
        compiler_params=pltpu.CompilerParams(
            dimension_semantics=("arbitrary", "arbitrary"),
            vmem_limit_bytes=100 * 1024 * 1024,
        ),
    )(offs, x_bf16, W1, W2, wtok)


NC = 4
CH = TPER // NC


def _combine_body(part_ref, out_ref, sx_ref, rx_ref, sy_ref, ry_ref,
                  send_sems, recv_sems):
    my_x = lax.axis_index("x")
    my_y = lax.axis_index("y")
    xnbr = (1 - my_x, my_y)
    ynbr = (my_x, 1 - my_y)
    other = 1 - my_x

    barrier = pltpu.get_barrier_semaphore()
    for nbr in (xnbr, ynbr):
        pl.semaphore_signal(barrier, inc=1, device_id=nbr,
                            device_id_type=pl.DeviceIdType.MESH)
    pl.semaphore_wait(barrier, 2)

    rdma_x = []
    for i in range(NC):
        sx_ref[pl.ds(i * CH, CH), :] = (
            part_ref[pl.ds(other * TPER + i * CH, CH), :].astype(jnp.bfloat16))
        r = pltpu.make_async_remote_copy(
            src_ref=sx_ref.at[pl.ds(i * CH, CH), :],
            dst_ref=rx_ref.at[pl.ds(i * CH, CH), :],
            send_sem=send_sems.at[0, i],
            recv_sem=recv_sems.at[0, i],
            device_id=xnbr,
            device_id_type=pl.DeviceIdType.MESH,
        )
        r.start()
        rdma_x.append(r)

    rdma_y = []
    for i in range(NC):
        rdma_x[i].wait_recv()
        s1 = (part_ref[pl.ds(my_x * TPER + i * CH, CH), :]
              + rx_ref[pl.ds(i * CH, CH), :].astype(jnp.float32))
        out_ref[pl.ds(i * CH, CH), :] = s1
        sy_ref[pl.ds(i * CH, CH), :] = s1.astype(jnp.bfloat16)
        r = pltpu.make_async_remote_copy(
            src_ref=sy_ref.at[pl.ds(i * CH, CH), :],
            dst_ref=ry_ref.at[pl.ds(i * CH, CH), :],
            send_sem=send_sems.at[1, i],
            recv_sem=recv_sems.at[1, i],
            device_id=ynbr,
            device_id_type=pl.DeviceIdType.MESH,
        )
        r.start()
        rdma_y.append(r)

    for i in range(NC):
        rdma_y[i].wait_recv()
        out_ref[pl.ds(i * CH, CH), :] += ry_ref[pl.ds(i * CH, CH), :].astype(
            jnp.float32)
    for i in range(NC):
        rdma_x[i].wait_send()
        rdma_y[i].wait_send()


def _combine(part):
    return pl.pallas_call(
        _combine_body,
        out_shape=jax.ShapeDtypeStruct((TPER, D), jnp.float32),
        in_specs=[pl.BlockSpec(memory_space=pltpu.VMEM)],
        out_specs=pl.BlockSpec(memory_space=pltpu.VMEM),
        scratch_shapes=[
            pltpu.VMEM((TPER, D), jnp.bfloat16),
            pltpu.VMEM((TPER, D), jnp.bfloat16),
            pltpu.VMEM((TPER, D), jnp.bfloat16),
            pltpu.VMEM((TPER, D), jnp.bfloat16),
            pltpu.SemaphoreType.DMA((2, NC)),
            pltpu.SemaphoreType.DMA((2, NC)),
        ],
        compiler_params=pltpu.CompilerParams(collective_id=1),
    )(part)


def kernel(x, router, W1, W2):
    my_x = lax.axis_index("x")
    my_y = lax.axis_index("y")

    xbf, gates = _dispatch(x, router.T)

    top2v, top2i = jax.lax.top_k(gates, 2)
    m = jnp.max(top2v, axis=1, keepdims=True)
    w = jnp.exp(top2v - m)
    w = w / jnp.sum(w, axis=1, keepdims=True)
    eids = jnp.arange(E)[None, :]
    wtok = (w[:, 0:1] * (top2i[:, 0:1] == eids)
            + w[:, 1:2] * (top2i[:, 1:2] == eids)).astype(jnp.float32)

    offs = jnp.stack([2 * my_y, 4 * my_x + 2 * my_y]).astype(jnp.int32)

    part = _ffn(offs, xbf, W1, W2, wtok)
    return _combine(part)


# device time: 68454 ns/iter; 1.6684x vs baseline; 1.1084x over previous
import jax
import jax.numpy as jnp
from jax import lax
from jax.experimental import pallas as pl
from jax.experimental.pallas import tpu as pltpu

T = 1024
D = 1024
E = 8
F = 2048
XS = 2
TPER = T // XS
EPER = E // XS
ELOC = EPER // 2
FB = 2
FBLK = F // FB


def _top2_weights(g):
    col = lax.broadcasted_iota(jnp.int32, g.shape, 1)
    m1 = jnp.max(g, axis=1, keepdims=True)
    i1 = jnp.min(jnp.where(g == m1, col, E), axis=1, keepdims=True)
    g2 = jnp.where(col == i1, -jnp.inf, g)
    m2 = jnp.max(g2, axis=1, keepdims=True)
    i2 = jnp.min(jnp.where(g2 == m2, col, E), axis=1, keepdims=True)
    t = jnp.exp(m2 - m1)
    w1v = 1.0 / (1.0 + t)
    w2v = t / (1.0 + t)
    return jnp.where(col == i1, w1v, 0.0) + jnp.where(col == i2, w2v, 0.0)


def _moe_body(off_ref, x_ref, rt_ref, w1_ref, w2_ref, out_ref,
              xbf_ref, rtfull_ref, gates_ref, wt_ref, send_sems, recv_sems):
    e = pl.program_id(0)
    fb = pl.program_id(1)
    my_x = lax.axis_index("x")
    my_y = lax.axis_index("y")
    xnbr = (1 - my_x, my_y)
    first = (e == 0) & (fb == 0)
    own_rows = pl.ds(my_x * TPER, TPER)
    rem_rows = pl.ds((1 - my_x) * TPER, TPER)

    rdma_rt = pltpu.make_async_remote_copy(
        src_ref=rt_ref, dst_ref=rtfull_ref.at[my_x],
        send_sem=send_sems.at[0], recv_sem=recv_sems.at[0],
        device_id=xnbr, device_id_type=pl.DeviceIdType.MESH)
    rdma_x = pltpu.make_async_remote_copy(
        src_ref=xbf_ref.at[own_rows, :], dst_ref=xbf_ref.at[own_rows, :],
        send_sem=send_sems.at[1], recv_sem=recv_sems.at[1],
        device_id=xnbr, device_id_type=pl.DeviceIdType.MESH)
    rdma_g = pltpu.make_async_remote_copy(
        src_ref=gates_ref.at[own_rows, :], dst_ref=gates_ref.at[own_rows, :],
        send_sem=send_sems.at[2], recv_sem=recv_sems.at[2],
        device_id=xnbr, device_id_type=pl.DeviceIdType.MESH)

    @pl.when(first)
    def _():
        barrier = pltpu.get_barrier_semaphore()
        pl.semaphore_signal(barrier, inc=1, device_id=xnbr,
                            device_id_type=pl.DeviceIdType.MESH)
        pl.semaphore_wait(barrier, 1)

        rtfull_ref[my_x] = rt_ref[...]
        rdma_rt.start()
        xbf_ref[own_rows, :] = x_ref[...].astype(jnp.bfloat16)
        rdma_x.start()
        rdma_rt.wait()
        gates_own = jax.lax.dot_general(
            x_ref[...], rtfull_ref[...].reshape(E, D),
            (((1,), (1,)), ((), ())),
            preferred_element_type=jnp.float32,
            precision=jax.lax.Precision.HIGHEST,
        )
        gates_ref[own_rows, :] = gates_own
        rdma_g.start()
        wt_ref[own_rows, :] = _top2_weights(gates_own)

    w1 = w1_ref[0].astype(jnp.bfloat16)
    w2 = w2_ref[0].astype(jnp.bfloat16)
    gid = off_ref[1] + e
    col = lax.broadcasted_iota(jnp.int32, (TPER, E), 1)

    def half(rows):
        xb = xbf_ref[rows, :]
        h = jnp.dot(xb, w1, preferred_element_type=jnp.float32)
        h = jnp.maximum(h, 0.0).astype(jnp.bfloat16)
        o = jnp.dot(h, w2, preferred_element_type=jnp.float32)
        wcol = jnp.sum(jnp.where(col == gid, wt_ref[rows, :], 0.0),
                       axis=1, keepdims=True)
        return o * wcol

    o_own = half(own_rows)

    @pl.when(first)
    def _():
        out_ref[own_rows, :] = o_own

    @pl.when(jnp.logical_not(first))
    def _():
        out_ref[own_rows, :] += o_own

    @pl.when(first)
    def _():
        rdma_x.wait_recv()
        rdma_g.wait_recv()
        wt_ref[rem_rows, :] = _top2_weights(gates_ref[rem_rows, :])
        rdma_x.wait_send()
        rdma_g.wait_send()

    o_rem = half(rem_rows)

    @pl.when(first)
    def _():
        out_ref[rem_rows, :] = o_rem

    @pl.when(jnp.logical_not(first))
    def _():
        out_ref[rem_rows, :] += o_rem


def _moe(offs, x, router_t, W1, W2):
    grid_spec = pltpu.PrefetchScalarGridSpec(
        num_scalar_prefetch=1,
        grid=(ELOC, FB),
        in_specs=[
            pl.BlockSpec((TPER, D), lambda e, fb, off: (0, 0)),
            pl.BlockSpec((EPER, D), lambda e, fb, off: (0, 0)),
            pl.BlockSpec((1, D, FBLK), lambda e, fb, off: (off[0] + e, 0, fb)),
            pl.BlockSpec((1, FBLK, D), lambda e, fb, off: (off[0] + e, fb, 0)),
        ],
        out_specs=pl.BlockSpec((T, D), lambda e, fb, off: (0, 0)),
        scratch_shapes=[
            pltpu.VMEM((T, D), jnp.bfloat16),
            pltpu.VMEM((XS, EPER, D), jnp.float32),
            pltpu.VMEM((T, E), jnp.float32),
            pltpu.VMEM((T, E), jnp.float32),
            pltpu.SemaphoreType.DMA((3,)),
            pltpu.SemaphoreType.DMA((3,)),
        ],
    )
    return pl.pallas_call(
        _moe_body,
        grid_spec=grid_spec,
        out_shape=jax.ShapeDtypeStruct((T, D), jnp.float32),
        compiler_params=pltpu.CompilerParams(
            dimension_semantics=("arbitrary", "arbitrary"),
            vmem_limit_bytes=100 * 1024 * 1024,
            collective_id=0,
        ),
    )(offs, x, router_t, W1, W2)


NC = 4
CH = TPER // NC


def _combine_body(part_ref, out_ref, sx_ref, rx_ref, sy_ref, ry_ref,
                  send_sems, recv_sems):
    my_x = lax.axis_index("x")
    my_y = lax.axis_index("y")
    xnbr = (1 - my_x, my_y)
    ynbr = (my_x, 1 - my_y)
    other = 1 - my_x

    barrier = pltpu.get_barrier_semaphore()
    for nbr in (xnbr, ynbr):
        pl.semaphore_signal(barrier, inc=1, device_id=nbr,
                            device_id_type=pl.DeviceIdType.MESH)
    pl.semaphore_wait(barrier, 2)

    rdma_x = []
    for i in range(NC):
        sx_ref[pl.ds(i * CH, CH), :] = (
            part_ref[pl.ds(other * TPER + i * CH, CH), :].astype(jnp.bfloat16))
        r = pltpu.make_async_remote_copy(
            src_ref=sx_ref.at[pl.ds(i * CH, CH), :],
            dst_ref=rx_ref.at[pl.ds(i * CH, CH), :],
            send_sem=send_sems.at[0, i],
            recv_sem=recv_sems.at[0, i],
            device_id=xnbr,
            device_id_type=pl.DeviceIdType.MESH,
        )
        r.start()
        rdma_x.append(r)

    rdma_y = []
    for i in range(NC):
        rdma_x[i].wait_recv()
        s1 = (part_ref[pl.ds(my_x * TPER + i * CH, CH), :]
              + rx_ref[pl.ds(i * CH, CH), :].astype(jnp.float32))
        out_ref[pl.ds(i * CH, CH), :] = s1
        sy_ref[pl.ds(i * CH, CH), :] = s1.astype(jnp.bfloat16)
        r = pltpu.make_async_remote_copy(
            src_ref=sy_ref.at[pl.ds(i * CH, CH), :],
            dst_ref=ry_ref.at[pl.ds(i * CH, CH), :],
            send_sem=send_sems.at[1, i],
            recv_sem=recv_sems.at[1, i],
            device_id=ynbr,
            device_id_type=pl.DeviceIdType.MESH,
        )
        r.start()
        rdma_y.append(r)

    for i in range(NC):
        rdma_y[i].wait_recv()
        out_ref[pl.ds(i * CH, CH), :] += ry_ref[pl.ds(i * CH, CH), :].astype(
            jnp.float32)
    for i in range(NC):
        rdma_x[i].wait_send()
        rdma_y[i].wait_send()


def _combine(part):
    return pl.pallas_call(
        _combine_body,
        out_shape=jax.ShapeDtypeStruct((TPER, D), jnp.float32),
        in_specs=[pl.BlockSpec(memory_space=pltpu.VMEM)],
        out_specs=pl.BlockSpec(memory_space=pltpu.VMEM),
        scratch_shapes=[
            pltpu.VMEM((TPER, D), jnp.bfloat16),
            pltpu.VMEM((TPER, D), jnp.bfloat16),
            pltpu.VMEM((TPER, D), jnp.bfloat16),
            pltpu.VMEM((TPER, D), jnp.bfloat16),
            pltpu.SemaphoreType.DMA((2, NC)),
            pltpu.SemaphoreType.DMA((2, NC)),
        ],
        compiler_params=pltpu.CompilerParams(collective_id=1),
    )(part)


def kernel(x, router, W1, W2):
    my_x = lax.axis_index("x")
    my_y = lax.axis_index("y")

    offs = jnp.stack([2 * my_y, 4 * my_x + 2 * my_y]).astype(jnp.int32)

    part = _moe(offs, x, router.T, W1, W2)
    return _combine(part)


# device time: 63422 ns/iter; 1.8008x vs baseline; 1.0793x over previous
import jax
import jax.numpy as jnp
from jax import lax
from jax.experimental import pallas as pl
from jax.experimental.pallas import tpu as pltpu

T = 1024
D = 1024
E = 8
F = 2048
XS = 2
TPER = T // XS
EPER = E // XS
ELOC = EPER // 2
FB = 2
FBLK = F // FB
XC = 2
XCH = TPER // XC
NC = 4
CH = TPER // NC


def _top2_weights(g):
    col = lax.broadcasted_iota(jnp.int32, g.shape, 1)
    m1 = jnp.max(g, axis=1, keepdims=True)
    i1 = jnp.min(jnp.where(g == m1, col, E), axis=1, keepdims=True)
    g2 = jnp.where(col == i1, -jnp.inf, g)
    m2 = jnp.max(g2, axis=1, keepdims=True)
    i2 = jnp.min(jnp.where(g2 == m2, col, E), axis=1, keepdims=True)
    t = jnp.exp(m2 - m1)
    w1v = 1.0 / (1.0 + t)
    w2v = t / (1.0 + t)
    return jnp.where(col == i1, w1v, 0.0) + jnp.where(col == i2, w2v, 0.0)


def _moe_body(off_ref, x_ref, rt_ref, w1_ref, w2_ref, out_ref,
              part_ref, xbf_ref, rtfull_ref, gates_ref, wt_ref,
              sx_ref, rx_ref, sy_ref, ry_ref,
              dsend, drecv, csend, crecv):
    e = pl.program_id(0)
    fb = pl.program_id(1)
    my_x = lax.axis_index("x")
    my_y = lax.axis_index("y")
    other = 1 - my_x
    xnbr = (other, my_y)
    ynbr = (my_x, 1 - my_y)
    first = (e == 0) & (fb == 0)
    last = (e == ELOC - 1) & (fb == FB - 1)
    own_rows = pl.ds(my_x * TPER, TPER)

    rdma_rt = pltpu.make_async_remote_copy(
        src_ref=rt_ref, dst_ref=rtfull_ref.at[my_x],
        send_sem=dsend.at[0], recv_sem=drecv.at[0],
        device_id=xnbr, device_id_type=pl.DeviceIdType.MESH)
    rdma_g = pltpu.make_async_remote_copy(
        src_ref=gates_ref.at[own_rows, :], dst_ref=gates_ref.at[own_rows, :],
        send_sem=dsend.at[1], recv_sem=drecv.at[1],
        device_id=xnbr, device_id_type=pl.DeviceIdType.MESH)
    rdma_xc = []
    for i in range(XC):
        sl = pl.ds(my_x * TPER + i * XCH, XCH)
        rdma_xc.append(pltpu.make_async_remote_copy(
            src_ref=xbf_ref.at[sl, :], dst_ref=xbf_ref.at[sl, :],
            send_sem=dsend.at[2 + i], recv_sem=drecv.at[2 + i],
            device_id=xnbr, device_id_type=pl.DeviceIdType.MESH))

    @pl.when(first)
    def _():
        barrier = pltpu.get_barrier_semaphore()
        for nbr in (xnbr, ynbr):
            pl.semaphore_signal(barrier, inc=1, device_id=nbr,
                                device_id_type=pl.DeviceIdType.MESH)
        pl.semaphore_wait(barrier, 2)

        rtfull_ref[my_x] = rt_ref[...]
        rdma_rt.start()
        xbf_ref[own_rows, :] = x_ref[...].astype(jnp.bfloat16)
        rdma_xc[0].start()
        rdma_rt.wait()
        gates_own = jax.lax.dot_general(
            x_ref[...], rtfull_ref[...].reshape(E, D),
            (((1,), (1,)), ((), ())),
            preferred_element_type=jnp.float32,
            precision=jax.lax.Precision.HIGHEST,
        )
        gates_ref[own_rows, :] = gates_own
        rdma_g.start()
        for i in range(1, XC):
            rdma_xc[i].start()
        wt_ref[own_rows, :] = _top2_weights(gates_own)

    w1 = w1_ref[0].astype(jnp.bfloat16)
    w2 = w2_ref[0].astype(jnp.bfloat16)
    gid = off_ref[1] + e

    def half_o(rows, n):
        xb = xbf_ref[rows, :]
        h = jnp.dot(xb, w1, preferred_element_type=jnp.float32)
        h = jnp.maximum(h, 0.0).astype(jnp.bfloat16)
        o = jnp.dot(h, w2, preferred_element_type=jnp.float32)
        col = lax.broadcasted_iota(jnp.int32, (n, E), 1)
        wcol = jnp.sum(jnp.where(col == gid, wt_ref[rows, :], 0.0),
                       axis=1, keepdims=True)
        return o * wcol

    o_own = half_o(own_rows, TPER)

    @pl.when(first)
    def _():
        part_ref[own_rows, :] = o_own

    @pl.when(jnp.logical_not(first))
    def _():
        part_ref[own_rows, :] += o_own

    @pl.when(first)
    def _():
        rdma_g.wait_recv()
        rem = pl.ds(other * TPER, TPER)
        wt_ref[rem, :] = _top2_weights(gates_ref[rem, :])
        for i in range(XC):
            rdma_xc[i].wait_recv()
            sl = pl.ds(other * TPER + i * XCH, XCH)
            part_ref[sl, :] = half_o(sl, XCH)
        for i in range(XC):
            rdma_xc[i].wait_send()
        rdma_g.wait_send()

    @pl.when(jnp.logical_not(first) & jnp.logical_not(last))
    def _():
        rem = pl.ds(other * TPER, TPER)
        part_ref[rem, :] += half_o(rem, TPER)

    cx = []
    cy = []
    for i in range(NC):
        cx.append(pltpu.make_async_remote_copy(
            src_ref=sx_ref.at[pl.ds(i * CH, CH), :],
            dst_ref=rx_ref.at[pl.ds(i * CH, CH), :],
            send_sem=csend.at[0, i], recv_sem=crecv.at[0, i],
            device_id=xnbr, device_id_type=pl.DeviceIdType.MESH))
        cy.append(pltpu.make_async_remote_copy(
            src_ref=sy_ref.at[pl.ds(i * CH, CH), :],
            dst_ref=ry_ref.at[pl.ds(i * CH, CH), :],
            send_sem=csend.at[1, i], recv_sem=crecv.at[1, i],
            device_id=ynbr, device_id_type=pl.DeviceIdType.MESH))

    @pl.when(last)
    def _():
        for i in range(NC):
            sl = pl.ds(other * TPER + i * CH, CH)
            pr = part_ref[sl, :] + half_o(sl, CH)
            sx_ref[pl.ds(i * CH, CH), :] = pr.astype(jnp.bfloat16)
            cx[i].start()
        for i in range(NC):
            cx[i].wait_recv()
            s1 = (part_ref[pl.ds(my_x * TPER + i * CH, CH), :]
                  + rx_ref[pl.ds(i * CH, CH), :].astype(jnp.float32))
            out_ref[pl.ds(i * CH, CH), :] = s1
            sy_ref[pl.ds(i * CH, CH), :] = s1.astype(jnp.bfloat16)
            cy[i].start()
        for i in range(NC):
            cy[i].wait_recv()
            out_ref[pl.ds(i * CH, CH), :] += (
                ry_ref[pl.ds(i * CH, CH), :].astype(jnp.float32))
        for i in range(NC):
            cx[i].wait_send()
            cy[i].wait_send()


def kernel(x, router, W1, W2):
    my_x = lax.axis_index("x")
    my_y = lax.axis_index("y")

    offs = jnp.stack([2 * my_y, 4 * my_x + 2 * my_y]).astype(jnp.int32)

    grid_spec = pltpu.PrefetchScalarGridSpec(
        num_scalar_prefetch=1,
        grid=(ELOC, FB),
        in_specs=[
            pl.BlockSpec((TPER, D), lambda e, fb, off: (0, 0)),
            pl.BlockSpec((EPER, D), lambda e, fb, off: (0, 0)),
            pl.BlockSpec((1, D, FBLK), lambda e, fb, off: (off[0] + e, 0, fb)),
            pl.BlockSpec((1, FBLK, D), lambda e, fb, off: (off[0] + e, fb, 0)),
        ],
        out_specs=pl.BlockSpec((TPER, D), lambda e, fb, off: (0, 0)),
        scratch_shapes=[
            pltpu.VMEM((T, D), jnp.float32),
            pltpu.VMEM((T, D), jnp.bfloat16),
            pltpu.VMEM((XS, EPER, D), jnp.float32),
            pltpu.VMEM((T, E), jnp.float32),
            pltpu.VMEM((T, E), jnp.float32),
            pltpu.VMEM((TPER, D), jnp.bfloat16),
            pltpu.VMEM((TPER, D), jnp.bfloat16),
            pltpu.VMEM((TPER, D), jnp.bfloat16),
            pltpu.VMEM((TPER, D), jnp.bfloat16),
            pltpu.SemaphoreType.DMA((2 + XC,)),
            pltpu.SemaphoreType.DMA((2 + XC,)),
            pltpu.SemaphoreType.DMA((2, NC)),
            pltpu.SemaphoreType.DMA((2, NC)),
        ],
    )
    return pl.pallas_call(
        _moe_body,
        grid_spec=grid_spec,
        out_shape=jax.ShapeDtypeStruct((TPER, D), jnp.float32),
        compiler_params=pltpu.CompilerParams(
            dimension_semantics=("arbitrary", "arbitrary"),
            vmem_limit_bytes=100 * 1024 * 1024,
            collective_id=0,
        ),
    )(offs, x, router.T, W1, W2)


# device time: 50843 ns/iter; 2.2463x vs baseline; 1.2474x over previous
import jax
import jax.numpy as jnp
from jax import lax
from jax.experimental import pallas as pl
from jax.experimental.pallas import tpu as pltpu

T = 1024
D = 1024
E = 8
F = 2048
XS = 2
TPER = T // XS
EPER = E // XS
ELOC = EPER // 2
FB = 2
FBLK = F // FB
NK = ELOC * FB
XC = 2
XCH = TPER // XC
NC = 4
CH = TPER // NC


def _top2_weights(g):
    col = lax.broadcasted_iota(jnp.int32, g.shape, 1)
    m1 = jnp.max(g, axis=1, keepdims=True)
    i1 = jnp.min(jnp.where(g == m1, col, E), axis=1, keepdims=True)
    g2 = jnp.where(col == i1, -jnp.inf, g)
    m2 = jnp.max(g2, axis=1, keepdims=True)
    i2 = jnp.min(jnp.where(g2 == m2, col, E), axis=1, keepdims=True)
    t = jnp.exp(m2 - m1)
    w1v = 1.0 / (1.0 + t)
    w2v = t / (1.0 + t)
    return jnp.where(col == i1, w1v, 0.0) + jnp.where(col == i2, w2v, 0.0)


def _moe_body(x_ref, rt_ref, w1_hbm, w2_hbm, out_ref,
              part_ref, xbf_ref, rtfull_ref, gates_ref, wt_ref,
              w1v_ref, w2v_ref,
              sx_ref, rx_ref, sy_ref, ry_ref,
              wsem1, wsem2, dsend, drecv, csend, crecv):
    my_x = lax.axis_index("x")
    my_y = lax.axis_index("y")
    other = 1 - my_x
    xnbr = (other, my_y)
    ynbr = (my_x, 1 - my_y)
    woff = 2 * my_y
    own_rows = pl.ds(my_x * TPER, TPER)
    rem = pl.ds(other * TPER, TPER)

    wdma = []
    for e in range(ELOC):
        for fb in range(FB):
            k = e * FB + fb
            c1 = pltpu.make_async_copy(
                w1_hbm.at[woff + e, :, pl.ds(fb * FBLK, FBLK)],
                w1v_ref.at[k], wsem1.at[k])
            c2 = pltpu.make_async_copy(
                w2_hbm.at[woff + e, pl.ds(fb * FBLK, FBLK), :],
                w2v_ref.at[k], wsem2.at[k])
            c1.start()
            c2.start()
            wdma.append((c1, c2))

    rdma_rt = pltpu.make_async_remote_copy(
        src_ref=rt_ref, dst_ref=rtfull_ref.at[my_x],
        send_sem=dsend.at[0], recv_sem=drecv.at[0],
        device_id=xnbr, device_id_type=pl.DeviceIdType.MESH)
    rdma_g = pltpu.make_async_remote_copy(
        src_ref=gates_ref.at[own_rows, :], dst_ref=gates_ref.at[own_rows, :],
        send_sem=dsend.at[1], recv_sem=drecv.at[1],
        device_id=xnbr, device_id_type=pl.DeviceIdType.MESH)
    rdma_xc = []
    for i in range(XC):
        sl = pl.ds(my_x * TPER + i * XCH, XCH)
        rdma_xc.append(pltpu.make_async_remote_copy(
            src_ref=xbf_ref.at[sl, :], dst_ref=xbf_ref.at[sl, :],
            send_sem=dsend.at[2 + i], recv_sem=drecv.at[2 + i],
            device_id=xnbr, device_id_type=pl.DeviceIdType.MESH))

    barrier = pltpu.get_barrier_semaphore()
    for nbr in (xnbr, ynbr):
        pl.semaphore_signal(barrier, inc=1, device_id=nbr,
                            device_id_type=pl.DeviceIdType.MESH)
    pl.semaphore_wait(barrier, 2)

    rtfull_ref[my_x] = rt_ref[...]
    rdma_rt.start()
    xbf_ref[own_rows, :] = x_ref[...].astype(jnp.bfloat16)
    rdma_xc[0].start()
    rdma_rt.wait()
    gates_own = jax.lax.dot_general(
        x_ref[...], rtfull_ref[...].reshape(E, D),
        (((1,), (1,)), ((), ())),
        preferred_element_type=jnp.float32,
        precision=jax.lax.Precision.HIGHEST,
    )
    gates_ref[own_rows, :] = gates_own
    rdma_g.start()
    for i in range(1, XC):
        rdma_xc[i].start()
    wt_ref[own_rows, :] = _top2_weights(gates_own)

    def half_o(k, rows, n):
        e = k // FB
        gid = 4 * my_x + 2 * my_y + e
        w1 = w1v_ref[k].astype(jnp.bfloat16)
        w2 = w2v_ref[k].astype(jnp.bfloat16)
        xb = xbf_ref[rows, :]
        h = jnp.dot(xb, w1, preferred_element_type=jnp.float32)
        h = jnp.maximum(h, 0.0).astype(jnp.bfloat16)
        o = jnp.dot(h, w2, preferred_element_type=jnp.float32)
        col = lax.broadcasted_iota(jnp.int32, (n, E), 1)
        wcol = jnp.sum(jnp.where(col == gid, wt_ref[rows, :], 0.0),
                       axis=1, keepdims=True)
        return o * wcol

    for k in range(NK - 1):
        wdma[k][0].wait()
        wdma[k][1].wait()
        o = half_o(k, own_rows, TPER)
        if k == 0:
            part_ref[own_rows, :] = o
        else:
            part_ref[own_rows, :] += o

    rdma_g.wait_recv()
    wt_ref[rem, :] = _top2_weights(gates_ref[rem, :])
    for i in range(XC):
        rdma_xc[i].wait_recv()
    wdma[NK - 1][0].wait()
    wdma[NK - 1][1].wait()
    for k in range(NK - 1):
        o = half_o(k, rem, TPER)
        if k == 0:
            part_ref[rem, :] = o
        else:
            part_ref[rem, :] += o

    cx = []
    cy = []
    for i in range(NC):
        cx.append(pltpu.make_async_remote_copy(
            src_ref=sx_ref.at[pl.ds(i * CH, CH), :],
            dst_ref=rx_ref.at[pl.ds(i * CH, CH), :],
            send_sem=csend.at[0, i], recv_sem=crecv.at[0, i],
            device_id=xnbr, device_id_type=pl.DeviceIdType.MESH))
        cy.append(pltpu.make_async_remote_copy(
            src_ref=sy_ref.at[pl.ds(i * CH, CH), :],
            dst_ref=ry_ref.at[pl.ds(i * CH, CH), :],
            send_sem=csend.at[1, i], recv_sem=crecv.at[1, i],
            device_id=ynbr, device_id_type=pl.DeviceIdType.MESH))

    for i in range(NC):
        sl = pl.ds(other * TPER + i * CH, CH)
        pr = part_ref[sl, :] + half_o(NK - 1, sl, CH)
        sx_ref[pl.ds(i * CH, CH), :] = pr.astype(jnp.bfloat16)
        cx[i].start()

    for i in range(NC):
        och = pl.ds(my_x * TPER + i * CH, CH)
        po = part_ref[och, :] + half_o(NK - 1, och, CH)
        cx[i].wait_recv()
        s1 = po + rx_ref[pl.ds(i * CH, CH), :].astype(jnp.float32)
        out_ref[pl.ds(i * CH, CH), :] = s1
        sy_ref[pl.ds(i * CH, CH), :] = s1.astype(jnp.bfloat16)
        cy[i].start()
    for i in range(NC):
        cy[i].wait_recv()
        out_ref[pl.ds(i * CH, CH), :] += (
            ry_ref[pl.ds(i * CH, CH), :].astype(jnp.float32))

    for i in range(XC):
        rdma_xc[i].wait_send()
    rdma_g.wait_send()
    for i in range(NC):
        cx[i].wait_send()
        cy[i].wait_send()


def kernel(x, router, W1, W2):
    return pl.pallas_call(
        _moe_body,
        out_shape=jax.ShapeDtypeStruct((TPER, D), jnp.float32),
        in_specs=[
            pl.BlockSpec(memory_space=pltpu.VMEM),
            pl.BlockSpec(memory_space=pltpu.VMEM),
            pl.BlockSpec(memory_space=pl.ANY),
            pl.BlockSpec(memory_space=pl.ANY),
        ],
        out_specs=pl.BlockSpec(memory_space=pltpu.VMEM),
        scratch_shapes=[
            pltpu.VMEM((T, D), jnp.float32),
            pltpu.VMEM((T, D), jnp.bfloat16),
            pltpu.VMEM((XS, EPER, D), jnp.float32),
            pltpu.VMEM((T, E), jnp.float32),
            pltpu.VMEM((T, E), jnp.float32),
            pltpu.VMEM((NK, D, FBLK), jnp.float32),
            pltpu.VMEM((NK, FBLK, D), jnp.float32),
            pltpu.VMEM((TPER, D), jnp.bfloat16),
            pltpu.VMEM((TPER, D), jnp.bfloat16),
            pltpu.VMEM((TPER, D), jnp.bfloat16),
            pltpu.VMEM((TPER, D), jnp.bfloat16),
            pltpu.SemaphoreType.DMA((NK,)),
            pltpu.SemaphoreType.DMA((NK,)),
            pltpu.SemaphoreType.DMA((2 + XC,)),
            pltpu.SemaphoreType.DMA((2 + XC,)),
            pltpu.SemaphoreType.DMA((2, NC)),
            pltpu.SemaphoreType.DMA((2, NC)),
        ],
        compiler_params=pltpu.CompilerParams(
            vmem_limit_bytes=100 * 1024 * 1024,
            collective_id=0,
        ),
    )(x, router.T, W1, W2)
